# baseline (device time: 29834 ns/iter reference)
import jax
import jax.numpy as jnp
from jax import lax
from jax.experimental import pallas as pl
from jax.experimental.pallas import tpu as pltpu

Y = 4


def kernel(x):
    m_per, n_tot = x.shape
    n_per = n_tot // Y

    def body(x_ref, out_ref, send_sems, recv_sems):
        ix = lax.axis_index("x")
        iy = lax.axis_index("y")
        iz = lax.axis_index("z")

        barrier_sem = pltpu.get_barrier_semaphore()
        for k in range(1, Y):
            peer = (iy + k) % Y
            pl.semaphore_signal(
                barrier_sem, inc=1,
                device_id=(ix, peer, iz),
                device_id_type=pl.DeviceIdType.MESH,
            )
        out_ref[pl.ds(iy * m_per, m_per), :] = x_ref[:, pl.ds(iy * n_per, n_per)]
        pl.semaphore_wait(barrier_sem, Y - 1)

        rdmas = []
        for k in range(1, Y):
            dst = (iy + k) % Y
            rdma = pltpu.make_async_remote_copy(
                src_ref=x_ref.at[:, pl.ds(dst * n_per, n_per)],
                dst_ref=out_ref.at[pl.ds(iy * m_per, m_per), :],
                send_sem=send_sems.at[k - 1],
                recv_sem=recv_sems.at[k - 1],
                device_id=(ix, dst, iz),
                device_id_type=pl.DeviceIdType.MESH,
            )
            rdma.start()
            rdmas.append(rdma)

        for k in range(1, Y):
            src = (iy - k) % Y
            recv = pltpu.make_async_remote_copy(
                src_ref=x_ref.at[:, pl.ds(src * n_per, n_per)],
                dst_ref=out_ref.at[pl.ds(src * m_per, m_per), :],
                send_sem=send_sems.at[k - 1],
                recv_sem=recv_sems.at[k - 1],
                device_id=(ix, src, iz),
                device_id_type=pl.DeviceIdType.MESH,
            )
            recv.wait_recv()

        for rdma in rdmas:
            rdma.wait_send()

    out_shape = jax.ShapeDtypeStruct((Y * m_per, n_per), jnp.bfloat16)
    return pl.pallas_call(
        body,
        out_shape=out_shape,
        in_specs=[pl.BlockSpec(memory_space=pltpu.VMEM)],
        out_specs=pl.BlockSpec(memory_space=pltpu.VMEM),
        scratch_shapes=[
            pltpu.SemaphoreType.DMA((Y - 1,)),
            pltpu.SemaphoreType.DMA((Y - 1,)),
        ],
        compiler_params=pltpu.CompilerParams(collective_id=0),
    )(x.astype(jnp.bfloat16))


# device time: 29498 ns/iter; 1.0114x vs baseline; 1.0114x over previous
import jax
import jax.numpy as jnp
from jax import lax
from jax.experimental import pallas as pl
from jax.experimental.pallas import tpu as pltpu

Y = 4


def kernel(x):
    m_per, n_tot = x.shape
    n_per = n_tot // Y

    def body(x_ref, out_ref, comm_ref, send_sems, recv_sems):
        ix = lax.axis_index("x")
        iy = lax.axis_index("y")
        iz = lax.axis_index("z")

        barrier_sem = pltpu.get_barrier_semaphore()
        for k in range(1, Y):
            peer = (iy + k) % Y
            pl.semaphore_signal(
                barrier_sem, inc=1,
                device_id=(ix, peer, iz),
                device_id_type=pl.DeviceIdType.MESH,
            )

        for k in range(1, Y):
            dst = (iy + k) % Y
            comm_ref[k - 1, :, :] = x_ref[:, pl.ds(dst * n_per, n_per)].astype(
                comm_ref.dtype
            )
        out_ref[pl.ds(iy * m_per, m_per), :] = x_ref[
            :, pl.ds(iy * n_per, n_per)
        ].astype(out_ref.dtype)

        pl.semaphore_wait(barrier_sem, Y - 1)

        rdmas = []
        for k in range(1, Y):
            dst = (iy + k) % Y
            rdma = pltpu.make_async_remote_copy(
                src_ref=comm_ref.at[k - 1],
                dst_ref=out_ref.at[pl.ds(iy * m_per, m_per), :],
                send_sem=send_sems.at[k - 1],
                recv_sem=recv_sems.at[k - 1],
                device_id=(ix, dst, iz),
                device_id_type=pl.DeviceIdType.MESH,
            )
            rdma.start()
            rdmas.append(rdma)

        for k in range(1, Y):
            src = (iy - k) % Y
            recv = pltpu.make_async_remote_copy(
                src_ref=comm_ref.at[k - 1],
                dst_ref=out_ref.at[pl.ds(src * m_per, m_per), :],
                send_sem=send_sems.at[k - 1],
                recv_sem=recv_sems.at[k - 1],
                device_id=(ix, src, iz),
                device_id_type=pl.DeviceIdType.MESH,
            )
            recv.wait_recv()

        for rdma in rdmas:
            rdma.wait_send()

    out_shape = jax.ShapeDtypeStruct((Y * m_per, n_per), jnp.bfloat16)
    return pl.pallas_call(
        body,
        out_shape=out_shape,
        in_specs=[pl.BlockSpec(memory_space=pltpu.VMEM)],
        out_specs=pl.BlockSpec(memory_space=pltpu.VMEM),
        scratch_shapes=[
            pltpu.VMEM((Y - 1, m_per, n_per), jnp.bfloat16),
            pltpu.SemaphoreType.DMA((Y - 1,)),
            pltpu.SemaphoreType.DMA((Y - 1,)),
        ],
        compiler_params=pltpu.CompilerParams(collective_id=0),
    )(x)
